# baseline (speedup 1.0000x reference)
import functools

import jax
import jax.numpy as jnp
from jax import lax
from jax.experimental import pallas as pl
from jax.experimental.pallas import tpu as pltpu
from jax.experimental.pallas import tpu_sc as plsc

_V = 256
_D = 4096
_B = 1024

_NC = 2
_NS = 16
_NW = _NC * _NS

_S = 256
_SC_BPW = _S // _NW

_TCR = _B - _S
_RT = 256
_CT = 512


def _sc_gather_body(table_hbm, idx_hbm, out_hbm, idx_v, rows_v, gsem, osem):
    wid = lax.axis_index("s") * _NC + lax.axis_index("c")
    base = wid * _SC_BPW
    pltpu.sync_copy(idx_hbm.at[pl.ds(base, _SC_BPW)], idx_v)
    pltpu.async_copy(table_hbm.at[idx_v], rows_v, gsem).wait()
    pltpu.async_copy(rows_v, out_hbm.at[pl.ds(base, _SC_BPW)], osem).wait()


def _sc_gather(sc_idx, embedding_weight):
    mesh = plsc.VectorSubcoreMesh(core_axis_name="c", subcore_axis_name="s")
    run = functools.partial(
        pl.kernel,
        mesh=mesh,
        out_type=jax.ShapeDtypeStruct((_S, _D), jnp.float32),
        scratch_types=[
            pltpu.VMEM((_SC_BPW,), jnp.int32),
            pltpu.VMEM((_SC_BPW, _D), jnp.float32),
            pltpu.SemaphoreType.DMA,
            pltpu.SemaphoreType.DMA,
        ],
    )(_sc_gather_body)
    return run(embedding_weight, sc_idx)


def _tc_body(idx_ref, table_ref, out_ref):
    vocab = lax.broadcasted_iota(jnp.int32, (_RT, _V), 1)
    onehot = (idx_ref[...] == vocab).astype(jnp.bfloat16)
    table = table_ref[...]
    hi = table.astype(jnp.bfloat16)
    lo = (table - hi.astype(jnp.float32)).astype(jnp.bfloat16)
    dims = (((1,), (0,)), ((), ()))
    acc = lax.dot_general(onehot, hi, dims,
                          preferred_element_type=jnp.float32)
    acc += lax.dot_general(onehot, lo, dims,
                           preferred_element_type=jnp.float32)
    out_ref[...] = acc


def _tc_onehot_matmul(tc_idx2, embedding_weight):
    return pl.pallas_call(
        _tc_body,
        grid=(_TCR // _RT, _D // _CT),
        in_specs=[
            pl.BlockSpec((_RT, 1), lambda i, j: (i, 0)),
            pl.BlockSpec((_V, _CT), lambda i, j: (0, j)),
        ],
        out_specs=pl.BlockSpec((_RT, _CT), lambda i, j: (i + _S // _RT, j)),
        out_shape=jax.ShapeDtypeStruct((_B, _D), jnp.float32),
    )(tc_idx2, embedding_weight)


@jax.jit
def _gather(indices_flat, embedding_weight):
    sc_out = _sc_gather(indices_flat[:_S], embedding_weight)
    tc_full = _tc_onehot_matmul(
        indices_flat[_S:].reshape(_TCR, 1), embedding_weight)
    return lax.dynamic_update_slice(tc_full, sc_out, (0, 0))


def kernel(indices, embedding_weight):
    b, n = indices.shape
    flat = indices.astype(jnp.int32).reshape(b * n)
    out = _gather(flat, embedding_weight)
    return out.reshape(b, n, _D)


# speedup vs baseline: 1.2979x; 1.2979x over previous
import functools

import jax
import jax.numpy as jnp
from jax import lax
from jax.experimental import pallas as pl
from jax.experimental.pallas import tpu as pltpu
from jax.experimental.pallas import tpu_sc as plsc

_V = 256
_D = 4096
_B = 1024

_NC = 2
_NS = 16
_NW = _NC * _NS

_S = 256
_SC_BPW = _S // _NW

_TCR = _B - _S
_RT = _TCR
_CT = 512


def _sc_gather_body(table_hbm, idx_hbm, out_hbm, idx_v, rows_v, gsem, osem):
    wid = lax.axis_index("s") * _NC + lax.axis_index("c")
    base = wid * _SC_BPW
    pltpu.sync_copy(idx_hbm.at[pl.ds(base, _SC_BPW)], idx_v)
    pltpu.async_copy(table_hbm.at[idx_v], rows_v, gsem).wait()
    pltpu.async_copy(rows_v, out_hbm.at[pl.ds(base, _SC_BPW)], osem).wait()


def _sc_gather(sc_idx, embedding_weight):
    mesh = plsc.VectorSubcoreMesh(core_axis_name="c", subcore_axis_name="s")
    run = functools.partial(
        pl.kernel,
        mesh=mesh,
        out_type=jax.ShapeDtypeStruct((_S, _D), jnp.float32),
        scratch_types=[
            pltpu.VMEM((_SC_BPW,), jnp.int32),
            pltpu.VMEM((_SC_BPW, _D), jnp.float32),
            pltpu.SemaphoreType.DMA,
            pltpu.SemaphoreType.DMA,
        ],
    )(_sc_gather_body)
    return run(embedding_weight, sc_idx)


def _tc_body(idx_ref, table_ref, out_ref):
    vocab = lax.broadcasted_iota(jnp.int32, (_RT, _V), 1)
    onehot = (idx_ref[...] == vocab).astype(jnp.bfloat16)
    table = table_ref[...]
    hi = table.astype(jnp.bfloat16)
    lo = (table - hi.astype(jnp.float32)).astype(jnp.bfloat16)
    dims = (((1,), (0,)), ((), ()))
    acc = lax.dot_general(onehot, hi, dims,
                          preferred_element_type=jnp.float32)
    acc += lax.dot_general(onehot, lo, dims,
                           preferred_element_type=jnp.float32)
    out_ref[...] = acc


def _tc_onehot_matmul(tc_idx2, embedding_weight):
    return pl.pallas_call(
        _tc_body,
        grid=(_D // _CT,),
        in_specs=[
            pl.BlockSpec((_RT, 1), lambda j: (0, 0)),
            pl.BlockSpec((_V, _CT), lambda j: (0, j)),
        ],
        out_specs=pl.BlockSpec((_RT, _CT), lambda j: (0, j)),
        out_shape=jax.ShapeDtypeStruct((_B, _D), jnp.float32),
    )(tc_idx2, embedding_weight)


@jax.jit
def _gather(indices_flat, embedding_weight):
    sc_out = _sc_gather(indices_flat[_TCR:], embedding_weight)
    tc_full = _tc_onehot_matmul(
        indices_flat[:_TCR].reshape(_TCR, 1), embedding_weight)
    return lax.dynamic_update_slice(tc_full, sc_out, (_TCR, 0))


def kernel(indices, embedding_weight):
    b, n = indices.shape
    flat = indices.astype(jnp.int32).reshape(b * n)
    out = _gather(flat, embedding_weight)
    return out.reshape(b, n, _D)


# speedup vs baseline: 1.3850x; 1.0670x over previous
import functools

import jax
import jax.numpy as jnp
from jax import lax
from jax.experimental import pallas as pl
from jax.experimental.pallas import tpu as pltpu
from jax.experimental.pallas import tpu_sc as plsc

_V = 256
_D = 4096
_B = 1024

_NC = 2
_NS = 16
_NW = _NC * _NS
_BPW = _B // _NW
_CHUNK = 8
_NCHUNK = _BPW // _CHUNK
_NBUF = 3


def _gather_kernel(table_hbm, idx_hbm, out_hbm, idx_v, rows_v, *sems):
    gsems, osems = sems[:_NBUF], sems[_NBUF:]
    wid = lax.axis_index("s") * _NC + lax.axis_index("c")
    base = wid * _BPW
    wpb = _V // _BPW
    pltpu.sync_copy(
        idx_hbm.at[wid // wpb, pl.ds((wid % wpb) * _BPW, _BPW)], idx_v)

    def fire_gather(c):
        return pltpu.async_copy(
            table_hbm.at[idx_v.at[pl.ds(c * _CHUNK, _CHUNK)]],
            rows_v.at[c % _NBUF], gsems[c % _NBUF])

    gathers = [None] * _NCHUNK
    outs = [None] * _NCHUNK
    for c in range(min(_NBUF, _NCHUNK)):
        gathers[c] = fire_gather(c)
    for c in range(_NCHUNK):
        buf = c % _NBUF
        gathers[c].wait()
        outs[c] = pltpu.async_copy(
            rows_v.at[buf], out_hbm.at[pl.ds(base + c * _CHUNK, _CHUNK)],
            osems[buf])
        nxt = c + _NBUF
        if nxt < _NCHUNK:
            outs[c].wait()
            gathers[nxt] = fire_gather(nxt)
    for c in range(max(0, _NCHUNK - _NBUF), _NCHUNK):
        outs[c].wait()


@jax.jit
def _gather(indices_2d, embedding_weight):
    mesh = plsc.VectorSubcoreMesh(core_axis_name="c", subcore_axis_name="s")
    run = functools.partial(
        pl.kernel,
        mesh=mesh,
        out_type=jax.ShapeDtypeStruct((_B, _D), jnp.float32),
        scratch_types=[
            pltpu.VMEM((_BPW,), jnp.int32),
            pltpu.VMEM((_NBUF, _CHUNK, _D), jnp.float32),
        ] + [pltpu.SemaphoreType.DMA] * (2 * _NBUF),
    )(_gather_kernel)
    return run(embedding_weight, indices_2d)


def kernel(indices, embedding_weight):
    b, n = indices.shape
    out = _gather(indices.astype(jnp.int32), embedding_weight)
    return out.reshape(b, n, _D)
